# Initial kernel scaffold; baseline (speedup 1.0000x reference)
import jax
import jax.numpy as jnp
from jax import lax
from jax.experimental import pallas as pl
from jax.experimental.pallas import tpu as pltpu
from jax.experimental.pallas import tpu_sc as plsc

N = 10000
E = 160000
D = 256
NC = 2
NS = 16
L = 16
HALF = D // NC
NP = 10240
CH = 128
NCHUNK = 80
EP = NS * NCHUNK * CH
TRASH = N
MAGIC = jnp.int32(0x5F3759DF)

_mesh = plsc.VectorSubcoreMesh(
    core_axis_name="c", subcore_axis_name="s", num_cores=NC, num_subcores=NS
)


def _rsqrt_newton(d):
    g = plsc.bitcast(MAGIC - (plsc.bitcast(d, jnp.int32) >> 1), jnp.float32)
    for _ in range(3):
        g = g * (1.5 - 0.5 * d * g * g)
    return g


def _deg_body(row_hbm, col_hbm, dis_hbm, rowv, colv, hist, zbuf, idxv, degv,
              disb, deg_sh):
    cid = lax.axis_index("c")
    sid = lax.axis_index("s")
    wid = sid * NC + cid
    pltpu.sync_copy(row_hbm.at[sid], rowv)
    pltpu.sync_copy(col_hbm.at[sid], colv)
    zero16 = jnp.zeros((L,), jnp.float32)

    def zhist(i, c):
        hist[i, :] = zero16
        return c

    lax.fori_loop(0, NP // L, zhist, 0)

    def zzbuf(i, c):
        zbuf[i, :] = zero16
        return c

    lax.fori_loop(0, (NP // L) // NS, zzbuf, 0)
    iota = lax.iota(jnp.int32, L)
    for t in range(5):
        for i in range(CH // L):
            idxv[t, pl.ds(i * L, L)] = iota + (t * CH + i * L)
    pltpu.sync_copy(zbuf, deg_sh.at[pl.ds(sid * 40, 40)])
    ones16 = jnp.ones((L,), jnp.float32)

    def chunk(j, c):
        for i in range(CH // L):
            r = rowv[j, pl.ds(i * L, L)]
            cc = colv[j, pl.ds(i * L, L)]
            plsc.addupdate_scatter(hist, [r >> 4, r & 15], ones16,
                                   mask=cc != TRASH)
        return c

    lax.fori_loop(0, NCHUNK, chunk, 0)
    plsc.subcore_barrier()
    for t in range(5):
        pltpu.sync_copy(hist.at[pl.ds(t * CH, CH)], deg_sh.at[idxv.at[t]],
                        add=True)
    plsc.subcore_barrier()
    pltpu.sync_copy(deg_sh.at[pl.ds(wid * 20, 20)], degv)

    def rr(i, c):
        disb[i, :] = _rsqrt_newton(degv[i, :] + 1.0)
        return c

    lax.fori_loop(0, 20, rr, 0)
    pltpu.sync_copy(disb, dis_hbm.at[pl.ds(wid * 20, 20)])


_deg_call = pl.kernel(
    _deg_body,
    out_type=jax.ShapeDtypeStruct((NP // L, L), jnp.float32),
    mesh=_mesh,
    scratch_types=[
        pltpu.VMEM((NCHUNK, CH), jnp.int32),
        pltpu.VMEM((NCHUNK, CH), jnp.int32),
        pltpu.VMEM((NP // L, L), jnp.float32),
        pltpu.VMEM(((NP // L) // NS, L), jnp.float32),
        pltpu.VMEM((5, CH), jnp.int32),
        pltpu.VMEM((20, L), jnp.float32),
        pltpu.VMEM((20, L), jnp.float32),
        pltpu.VMEM_SHARED((NP // L, L), jnp.float32),
    ],
)


def _y_body(x_ref, dis_ref, y_ref):
    y_ref[...] = x_ref[...] * dis_ref[...]


_NB = 10
_R = N // _NB


def _y_call(x, dis2d):
    return pl.pallas_call(
        _y_body,
        grid=(NC, _NB),
        in_specs=[
            pl.BlockSpec((_R, HALF), lambda h, b: (b, h)),
            pl.BlockSpec((_R, 1), lambda h, b: (b, 0)),
        ],
        out_specs=pl.BlockSpec((_R, HALF), lambda h, b: (h * _NB + b, 0)),
        out_shape=jax.ShapeDtypeStruct((NC * N, HALF), jnp.float32),
    )(x, dis2d)


def _main_body(y_hbm, row_hbm, col_hbm, dis_hbm, out_hbm, rowv, colv, gbuf,
               obuf, disv, acc_sh, sem0, sem1):
    cid = lax.axis_index("c")
    sid = lax.axis_index("s")
    wid2 = cid * NS + sid
    pltpu.sync_copy(row_hbm.at[wid2], rowv)
    pltpu.sync_copy(col_hbm.at[sid], colv)
    pltpu.sync_copy(dis_hbm.at[pl.ds(sid * 40, 40)], disv)
    RPT = N // NS
    pltpu.sync_copy(y_hbm.at[pl.ds(cid * N + sid * RPT, RPT)],
                    acc_sh.at[pl.ds(sid * RPT, RPT)])
    zero16 = jnp.zeros((L,), jnp.float32)

    def zob(i, c):
        for k in range(HALF // L):
            obuf[i, pl.ds(k * L, L)] = zero16
        return c

    lax.fori_loop(0, 15, zob, 0)
    pltpu.sync_copy(obuf.at[pl.ds(0, 15)],
                    acc_sh.at[pl.ds(N + sid * 15, 15)])
    plsc.subcore_barrier()
    sems = (sem0, sem1)

    def issue(j, b):
        pltpu.async_copy(y_hbm.at[rowv.at[j]], gbuf.at[b], sems[b])

    def wait(b):
        pltpu.make_async_copy(y_hbm.at[pl.ds(0, CH)], gbuf.at[b],
                              sems[b]).wait()

    issue(0, 0)
    issue(1, 1)

    def edge_loop(t, c):
        for b in range(2):
            j = 2 * t + b
            wait(b)
            pltpu.sync_copy(gbuf.at[b], acc_sh.at[colv.at[j]], add=True)
            issue(j + 2, b)
        return c

    lax.fori_loop(0, NCHUNK // 2 - 1, edge_loop, 0)
    for b in range(2):
        j = NCHUNK - 2 + b
        wait(b)
        pltpu.sync_copy(gbuf.at[b], acc_sh.at[colv.at[j]], add=True)
    plsc.subcore_barrier()

    def blk(bki, c):
        base = sid * 640 + bki * 64
        pltpu.sync_copy(acc_sh.at[pl.ds(base, 64)], obuf)

        def rowfn(i, c2):
            loc = bki * 64 + i
            dvec = plsc.load_gather(
                disv,
                [jnp.full((L,), loc >> 4, jnp.int32),
                 jnp.full((L,), loc & 15, jnp.int32)])
            for k in range(HALF // L):
                obuf[i, pl.ds(k * L, L)] = obuf[i, pl.ds(k * L, L)] * dvec
            return c2

        lax.fori_loop(0, 64, rowfn, 0)
        pltpu.sync_copy(obuf, out_hbm.at[pl.ds(cid * NP + base, 64)])
        return c

    lax.fori_loop(0, 10, blk, 0)


_main_call = pl.kernel(
    _main_body,
    out_type=jax.ShapeDtypeStruct((NC * NP, HALF), jnp.float32),
    mesh=_mesh,
    scratch_types=[
        pltpu.VMEM((NCHUNK, CH), jnp.int32),
        pltpu.VMEM((NCHUNK, CH), jnp.int32),
        pltpu.VMEM((2, CH, HALF), jnp.float32),
        pltpu.VMEM((64, HALF), jnp.float32),
        pltpu.VMEM((40, L), jnp.float32),
        pltpu.VMEM_SHARED((NP, HALF), jnp.float32),
        pltpu.SemaphoreType.DMA,
        pltpu.SemaphoreType.DMA,
    ],
)


def kernel(x, edge_index):
    row = edge_index[0].astype(jnp.int32)
    col = edge_index[1].astype(jnp.int32)
    col = jnp.where(row == col, TRASH, col)
    pad = EP - E
    row_p = jnp.concatenate([row, jnp.zeros((pad,), jnp.int32)])
    col_p = jnp.concatenate([col, jnp.full((pad,), TRASH, jnp.int32)])
    row3 = row_p.reshape(NS, NCHUNK, CH)
    col3 = col_p.reshape(NS, NCHUNK, CH)
    row2 = (row_p[None, :]
            + (jnp.arange(NC, dtype=jnp.int32) * N)[:, None]).reshape(
                NC * NS, NCHUNK, CH)
    dis2 = _deg_call(row3, col3)
    dis2d = dis2.reshape(NP)[:N].reshape(N, 1)
    y2 = _y_call(x, dis2d)
    out_flat = _main_call(y2, row2, col3, dis2)
    out = out_flat.reshape(NC, NP, HALF)[:, :N]
    return out.transpose(1, 0, 2).reshape(N, D)


# speedup vs baseline: 8.0599x; 8.0599x over previous
import jax
import jax.numpy as jnp
from jax import lax
from jax.experimental import pallas as pl
from jax.experimental.pallas import tpu as pltpu
from jax.experimental.pallas import tpu_sc as plsc

N = 10000
E = 160000
D = 256
NC = 2
NS = 16
L = 16
HALF = D // NC
NP = 10240
CH = 128
NG = 10
NCHUNK = NG * 8
EP = NS * NCHUNK * CH
TRASH = N

_mesh = plsc.VectorSubcoreMesh(
    core_axis_name="c", subcore_axis_name="s", num_cores=NC, num_subcores=NS
)


def _deg_body(row_hbm, col_hbm, deg_hbm, ricb, cicb, wbuf, zbuf, deg_sh):
    cid = lax.axis_index("c")
    sid = lax.axis_index("s")
    zero16 = jnp.zeros((L,), jnp.float32)

    def zz(i, c):
        zbuf[pl.ds(i * L, L)] = zero16
        return c

    lax.fori_loop(0, 640 // L, zz, 0)
    pltpu.sync_copy(zbuf, deg_sh.at[pl.ds(sid * 640, 640)])
    plsc.subcore_barrier()

    def group(g, c):
        pltpu.sync_copy(row_hbm.at[sid * NG + g], ricb)
        pltpu.sync_copy(col_hbm.at[sid * NG + g], cicb)

        def ch_fn(r, c2):
            for i in range(CH // L):
                cc = cicb[r, pl.ds(i * L, L)]
                wbuf[pl.ds(i * L, L)] = jnp.where(cc != TRASH, 1.0, 0.0)
            pltpu.sync_copy(wbuf, deg_sh.at[ricb.at[r]], add=True)
            return c2

        lax.fori_loop(0, 8, ch_fn, 0)
        return c

    lax.fori_loop(0, NG, group, 0)
    plsc.subcore_barrier()

    @pl.when(cid == 0)
    def _():
        pltpu.sync_copy(deg_sh.at[pl.ds(sid * 640, 640)], zbuf)
        pltpu.sync_copy(zbuf, deg_hbm.at[pl.ds(sid * 640, 640)])


_deg_call = pl.kernel(
    _deg_body,
    out_type=jax.ShapeDtypeStruct((NP,), jnp.float32),
    mesh=_mesh,
    scratch_types=[
        pltpu.VMEM((8, CH), jnp.int32),
        pltpu.VMEM((8, CH), jnp.int32),
        pltpu.VMEM((CH,), jnp.float32),
        pltpu.VMEM((640,), jnp.float32),
        pltpu.VMEM_SHARED((NP,), jnp.float32),
    ],
)


_R = 640
_NB = NP // _R


def _y_body(x_ref, degn_ref, y_ref):
    y_ref[...] = x_ref[...] * lax.rsqrt(degn_ref[...] + 1.0)


def _y_call(x, degn):
    return pl.pallas_call(
        _y_body,
        grid=(NC, _NB),
        in_specs=[
            pl.BlockSpec((_R, HALF), lambda h, b: (b, h)),
            pl.BlockSpec((_R, 1), lambda h, b: (b, 0)),
        ],
        out_specs=pl.BlockSpec((_R, HALF), lambda h, b: (h * _NB + b, 0)),
        out_shape=jax.ShapeDtypeStruct((NC * NP, HALF), jnp.float32),
    )(x, degn)


def _main_body(y_hbm, row_hbm, col_hbm, out_hbm, ricb, cicb, gbuf, acc_sh):
    cid = lax.axis_index("c")
    sid = lax.axis_index("s")
    wid2 = cid * NS + sid
    pltpu.sync_copy(y_hbm.at[pl.ds(cid * NP + sid * 640, 640)],
                    acc_sh.at[pl.ds(sid * 640, 640)])
    plsc.subcore_barrier()

    def group(g, c):
        pltpu.sync_copy(row_hbm.at[wid2 * NG + g], ricb)
        pltpu.sync_copy(col_hbm.at[sid * NG + g], cicb)

        def ch_fn(r, c2):
            pltpu.sync_copy(y_hbm.at[ricb.at[r]], gbuf)
            pltpu.sync_copy(gbuf, acc_sh.at[cicb.at[r]], add=True)
            return c2

        lax.fori_loop(0, 8, ch_fn, 0)
        return c

    lax.fori_loop(0, NG, group, 0)
    plsc.subcore_barrier()
    pltpu.sync_copy(acc_sh.at[pl.ds(sid * 640, 640)],
                    out_hbm.at[pl.ds(cid * NP + sid * 640, 640)])


_main_call = pl.kernel(
    _main_body,
    out_type=jax.ShapeDtypeStruct((NC * NP, HALF), jnp.float32),
    mesh=_mesh,
    scratch_types=[
        pltpu.VMEM((8, CH), jnp.int32),
        pltpu.VMEM((8, CH), jnp.int32),
        pltpu.VMEM((CH, HALF), jnp.float32),
        pltpu.VMEM_SHARED((NP, HALF), jnp.float32),
    ],
)


def _scale_body(acc_ref, degn_ref, out_ref):
    out_ref[...] = acc_ref[...] * lax.rsqrt(degn_ref[...] + 1.0)


def _scale_call(acc, degn):
    return pl.pallas_call(
        _scale_body,
        grid=(NC, _NB),
        in_specs=[
            pl.BlockSpec((_R, HALF), lambda h, b: (h * _NB + b, 0)),
            pl.BlockSpec((_R, 1), lambda h, b: (b, 0)),
        ],
        out_specs=pl.BlockSpec((_R, HALF), lambda h, b: (b, h)),
        out_shape=jax.ShapeDtypeStruct((NP, D), jnp.float32),
    )(acc, degn)


def kernel(x, edge_index):
    row = edge_index[0].astype(jnp.int32)
    col = edge_index[1].astype(jnp.int32)
    col = jnp.where(row == col, TRASH, col)
    pad = EP - E
    row_p = jnp.concatenate([row, jnp.zeros((pad,), jnp.int32)])
    col_p = jnp.concatenate([col, jnp.full((pad,), TRASH, jnp.int32)])
    row3 = row_p.reshape(NS * NG, 8, CH)
    col3 = col_p.reshape(NS * NG, 8, CH)
    row2 = (row_p[None, :]
            + (jnp.arange(NC, dtype=jnp.int32) * NP)[:, None]).reshape(
                NC * NS * NG, 8, CH)
    deg1 = _deg_call(row3, col3)
    degn = deg1.reshape(NP, 1)
    y2 = _y_call(x, degn)
    acc = _main_call(y2, row2, col3)
    return _scale_call(acc, degn)[:N]


# speedup vs baseline: 10.6378x; 1.3198x over previous
import jax
import jax.numpy as jnp
from jax import lax
from jax.experimental import pallas as pl
from jax.experimental.pallas import tpu as pltpu
from jax.experimental.pallas import tpu_sc as plsc

N = 10000
E = 160000
D = 256
NC = 2
NS = 16
L = 16
HALF = D // NC
NP = 10240
CH = 128
NG = 10
NCHUNK = NG * 8
EP = NS * NCHUNK * CH
TRASH = N

_mesh = plsc.VectorSubcoreMesh(
    core_axis_name="c", subcore_axis_name="s", num_cores=NC, num_subcores=NS
)


def _deg_body(row_hbm, col_hbm, deg_hbm, ricb, cicb, wbuf, zbuf, deg_sh):
    cid = lax.axis_index("c")
    sid = lax.axis_index("s")
    zero16 = jnp.zeros((L,), jnp.float32)

    def zz(i, c):
        zbuf[pl.ds(i * L, L)] = zero16
        return c

    lax.fori_loop(0, 640 // L, zz, 0)
    pltpu.sync_copy(zbuf, deg_sh.at[pl.ds(sid * 640, 640)])
    plsc.subcore_barrier()

    def group(g, c):
        pltpu.sync_copy(row_hbm.at[sid * NG + g], ricb)
        pltpu.sync_copy(col_hbm.at[sid * NG + g], cicb)

        def ch_fn(r, c2):
            for i in range(CH // L):
                cc = cicb[r, pl.ds(i * L, L)]
                wbuf[pl.ds(i * L, L)] = jnp.where(cc != TRASH, 1.0, 0.0)
            pltpu.sync_copy(wbuf, deg_sh.at[ricb.at[r]], add=True)
            return c2

        lax.fori_loop(0, 8, ch_fn, 0)
        return c

    lax.fori_loop(0, NG, group, 0)
    plsc.subcore_barrier()

    @pl.when(cid == 0)
    def _():
        pltpu.sync_copy(deg_sh.at[pl.ds(sid * 640, 640)], zbuf)
        pltpu.sync_copy(zbuf, deg_hbm.at[pl.ds(sid * 640, 640)])


_deg_call = pl.kernel(
    _deg_body,
    out_type=jax.ShapeDtypeStruct((NP,), jnp.float32),
    mesh=_mesh,
    scratch_types=[
        pltpu.VMEM((8, CH), jnp.int32),
        pltpu.VMEM((8, CH), jnp.int32),
        pltpu.VMEM((CH,), jnp.float32),
        pltpu.VMEM((640,), jnp.float32),
        pltpu.VMEM_SHARED((NP,), jnp.float32),
    ],
)


_R = 640
_NB = NP // _R


def _y_body(x_ref, degn_ref, y_ref):
    y_ref[...] = x_ref[...] * lax.rsqrt(degn_ref[...] + 1.0)


def _y_call(x, degn):
    return pl.pallas_call(
        _y_body,
        grid=(NC, _NB),
        in_specs=[
            pl.BlockSpec((_R, HALF), lambda h, b: (b, h)),
            pl.BlockSpec((_R, 1), lambda h, b: (b, 0)),
        ],
        out_specs=pl.BlockSpec((_R, HALF), lambda h, b: (h * _NB + b, 0)),
        out_shape=jax.ShapeDtypeStruct((NC * NP, HALF), jnp.float32),
    )(x, degn)


def _main_body(y_hbm, row_hbm, col_hbm, out_hbm, ricb, cicb, gbuf, acc_sh,
               semg0, semg1, sems0, sems1, semi):
    cid = lax.axis_index("c")
    sid = lax.axis_index("s")
    wid2 = cid * NS + sid
    semg = (semg0, semg1)
    sems = (sems0, sems1)
    pltpu.sync_copy(y_hbm.at[pl.ds(cid * NP + sid * 640, 640)],
                    acc_sh.at[pl.ds(sid * 640, 640)])
    plsc.subcore_barrier()

    pltpu.sync_copy(row_hbm.at[wid2 * NG], ricb.at[0])
    pltpu.sync_copy(col_hbm.at[sid * NG], cicb.at[0])
    pltpu.async_copy(row_hbm.at[wid2 * NG + 1], ricb.at[1], semi)
    pltpu.async_copy(col_hbm.at[sid * NG + 1], cicb.at[1], semi)
    pltpu.async_copy(y_hbm.at[ricb.at[0, 0]], gbuf.at[0], semg0)
    pltpu.async_copy(y_hbm.at[ricb.at[0, 1]], gbuf.at[1], semg1)

    def body(t, c):
        for b in range(2):
            j = 2 * t + b
            g = j // 8
            r = j % 8
            ib = g % 2

            @pl.when(jnp.logical_and(r == 6, g < NG - 1))
            def _():
                pltpu.make_async_copy(row_hbm.at[0], ricb.at[0], semi).wait()
                pltpu.make_async_copy(col_hbm.at[0], cicb.at[0], semi).wait()

            pltpu.make_async_copy(y_hbm.at[pl.ds(0, CH)], gbuf.at[b],
                                  semg[b]).wait()
            pltpu.async_copy(gbuf.at[b], acc_sh.at[cicb.at[ib, r]], sems[b],
                             add=True)
            pltpu.make_async_copy(gbuf.at[b], acc_sh.at[pl.ds(0, CH)],
                                  sems[b]).wait()

            @pl.when(j < NCHUNK - 2)
            def _():
                j2 = j + 2
                ib2 = (j2 // 8) % 2
                r2 = j2 % 8
                pltpu.async_copy(y_hbm.at[ricb.at[ib2, r2]], gbuf.at[b],
                                 semg[b])

            @pl.when(jnp.logical_and(r == 7, g < NG - 2))
            def _():
                pltpu.async_copy(row_hbm.at[wid2 * NG + g + 2], ricb.at[ib],
                                 semi)
                pltpu.async_copy(col_hbm.at[sid * NG + g + 2], cicb.at[ib],
                                 semi)
        return c

    lax.fori_loop(0, NCHUNK // 2, body, 0)
    plsc.subcore_barrier()
    pltpu.sync_copy(acc_sh.at[pl.ds(sid * 640, 640)],
                    out_hbm.at[pl.ds(cid * NP + sid * 640, 640)])


_main_call = pl.kernel(
    _main_body,
    out_type=jax.ShapeDtypeStruct((NC * NP, HALF), jnp.float32),
    mesh=_mesh,
    scratch_types=[
        pltpu.VMEM((2, 8, CH), jnp.int32),
        pltpu.VMEM((2, 8, CH), jnp.int32),
        pltpu.VMEM((2, CH, HALF), jnp.float32),
        pltpu.VMEM_SHARED((NP, HALF), jnp.float32),
        pltpu.SemaphoreType.DMA,
        pltpu.SemaphoreType.DMA,
        pltpu.SemaphoreType.DMA,
        pltpu.SemaphoreType.DMA,
        pltpu.SemaphoreType.DMA,
    ],
)


def _scale_body(acc_ref, degn_ref, out_ref):
    out_ref[...] = acc_ref[...] * lax.rsqrt(degn_ref[...] + 1.0)


def _scale_call(acc, degn):
    return pl.pallas_call(
        _scale_body,
        grid=(NC, _NB),
        in_specs=[
            pl.BlockSpec((_R, HALF), lambda h, b: (h * _NB + b, 0)),
            pl.BlockSpec((_R, 1), lambda h, b: (b, 0)),
        ],
        out_specs=pl.BlockSpec((_R, HALF), lambda h, b: (b, h)),
        out_shape=jax.ShapeDtypeStruct((N, D), jnp.float32),
    )(acc, degn)


def kernel(x, edge_index):
    row = edge_index[0].astype(jnp.int32)
    col = edge_index[1].astype(jnp.int32)
    col = jnp.where(row == col, TRASH, col)
    pad = EP - E
    row_p = jnp.concatenate([row, jnp.zeros((pad,), jnp.int32)])
    col_p = jnp.concatenate([col, jnp.full((pad,), TRASH, jnp.int32)])
    row3 = row_p.reshape(NS * NG, 8, CH)
    col3 = col_p.reshape(NS * NG, 8, CH)
    row2 = (row_p[None, :]
            + (jnp.arange(NC, dtype=jnp.int32) * NP)[:, None]).reshape(
                NC * NS * NG, 8, CH)
    deg1 = _deg_call(row3, col3)
    degn = deg1.reshape(NP, 1)
    y2 = _y_call(x, degn)
    acc = _main_call(y2, row2, col3)
    return _scale_call(acc, degn)
